# Initial kernel scaffold; baseline (speedup 1.0000x reference)
import jax
import jax.numpy as jnp
from jax.experimental import pallas as pl


def kernel(coordinates, positions, elem_table, res_table, pos_W, pos_b, edge_W1, edge_b1, edge_W2, edge_b2, Wm, Ws, Wa, bl, Wo, bo, elements, residues, edge_index, residue_ids):
    raise NotImplementedError("write your pallas kernel here")


# speedup vs baseline: 1.0000x; 1.0000x over previous
import jax
import jax.numpy as jnp
from jax.experimental import pallas as pl

_N = 100000
_HIDDEN = 16
_D = 3 * _HIDDEN
_EDGE_DIM = 16
_LAYERS = 2
_NUM_RES = 5000


def _sinusoidal(n, dim):
    pos = jnp.arange(n, dtype=jnp.float32)[:, None]
    i = jnp.arange(dim // 2, dtype=jnp.float32)[None, :]
    div = jnp.exp(-jnp.log(10000.0) * (2.0 * i) / dim)
    ang = pos * div
    pe = jnp.zeros((n, dim), dtype=jnp.float32)
    pe = pe.at[:, 0::2].set(jnp.sin(ang))
    pe = pe.at[:, 1::2].set(jnp.cos(ang))
    return pe


def kernel(coordinates, positions, elem_table, res_table, pos_W, pos_b, edge_W1, edge_b1, edge_W2, edge_b2, Wm, Ws, Wa, bl, Wo, bo, elements, residues, edge_index, residue_ids):
    src = edge_index[0]
    dst = edge_index[1]
    diff = coordinates[src] - coordinates[dst]
    dist = jnp.sqrt(jnp.sum(diff * diff, axis=-1) + 1e-12)
    centers = jnp.linspace(0.0, 10.0, _EDGE_DIM)
    rbf = jnp.exp(-10.0 * (dist[:, None] - centers[None, :]) ** 2)
    e_emb = jax.nn.relu(rbf @ edge_W1 + edge_b1) @ edge_W2 + edge_b2
    h = jnp.concatenate([
        jnp.take(elem_table, elements, axis=0),
        jnp.take(res_table, residues, axis=0),
        positions[:, None] @ pos_W + pos_b,
    ], axis=-1)
    h = h + _sinusoidal(_N, _D)
    for l in range(_LAYERS):
        m = jax.nn.relu(jnp.take(h, src, axis=0) @ Wm[l] + e_emb)
        agg = jax.ops.segment_sum(m, dst, num_segments=_N)
        h = jax.nn.relu(h @ Ws[l] + agg @ Wa[l] + bl[l])
    out_nodes = h @ Wo + bo
    sums = jax.ops.segment_sum(out_nodes, residue_ids, num_segments=_NUM_RES)
    counts = jax.ops.segment_sum(jnp.ones((_N,), dtype=jnp.float32), residue_ids, num_segments=_NUM_RES)

    def _div_body(s_ref, c_ref, o_ref):
        o_ref[...] = s_ref[...] / jnp.maximum(c_ref[...], 1.0)

    return pl.pallas_call(
        _div_body,
        out_shape=jax.ShapeDtypeStruct((_NUM_RES, 2), jnp.float32),
    )(sums, counts[:, None])


# speedup vs baseline: 3.2515x; 3.2514x over previous
import functools

import jax
import jax.numpy as jnp
from jax import lax
from jax.experimental import pallas as pl
from jax.experimental.pallas import tpu as pltpu
from jax.experimental.pallas import tpu_sc as plsc

_N = 100000
_E = 1600000
_HIDDEN = 16
_D = 48
_EDGE_DIM = 16
_EDGE_HID = 32
_NUM_RES = 5000

_C = 16
_NCHUNK = 3
_NTILES = 32
_BLK = 1024
_SUB = 128
_EBLKS = 49
_E_PAD = _NTILES * _BLK * _EBLKS
_IDXROWS = _E_PAD // _SUB
_ROWS_PER_TILE = _IDXROWS // _NTILES
_NB = 800
_N_PAD = 102400
_RIDROWS = _N_PAD // _SUB

_MESH = dict(core_axis_name="c", subcore_axis_name="s", num_cores=2,
             num_subcores=16)
_SC_PARAMS = pltpu.CompilerParams(use_tc_tiling_on_sc=False)
_NEG = -1e30


def _sinusoidal(n, dim):
    pos = jnp.arange(n, dtype=jnp.float32)[:, None]
    i = jnp.arange(dim // 2, dtype=jnp.float32)[None, :]
    div = jnp.exp(-jnp.log(10000.0) * (2.0 * i) / dim)
    ang = pos * div
    pe = jnp.zeros((n, dim), dtype=jnp.float32)
    pe = pe.at[:, 0::2].set(jnp.sin(ang))
    pe = pe.at[:, 1::2].set(jnp.cos(ang))
    return pe


def _t1_body(el_ref, rs_ref, ps_ref, et_ref, rt_ref, pw_ref, pb_ref, pe_ref,
             wm_ref, h_ref, w0_ref, w1_ref, w2_ref):
    el = el_ref[...]
    rs = rs_ref[...]
    e_emb = jnp.zeros((_NB, _HIDDEN), jnp.float32)
    for k in range(5):
        e_emb = e_emb + jnp.where(el == k, 1.0, 0.0) * et_ref[k:k + 1, :]
    r_emb = jnp.zeros((_NB, _HIDDEN), jnp.float32)
    for k in range(4):
        r_emb = r_emb + jnp.where(rs == k, 1.0, 0.0) * rt_ref[k:k + 1, :]
    p_emb = ps_ref[...] * pw_ref[...] + pb_ref[...]
    h = jnp.concatenate([e_emb, r_emb, p_emb], axis=1) + pe_ref[...]
    h_ref[...] = h
    hw = jnp.dot(h, wm_ref[...])
    w0_ref[...] = hw[:, 0:16]
    w1_ref[...] = hw[:, 16:32]
    w2_ref[...] = hw[:, 32:48]


def _node_features(elements, residues, positions, elem_table, res_table,
                   pos_W, pos_b, pe, Wm0):
    nb = pl.BlockSpec((_NB, 1), lambda i: (i, 0))
    full = lambda s: pl.BlockSpec(s, lambda i: tuple(0 for _ in s))
    cblk = pl.BlockSpec((_NB, _C), lambda i: (i, 0))
    return pl.pallas_call(
        _t1_body,
        grid=(_N // _NB,),
        in_specs=[nb, nb, nb, full((5, 16)), full((4, 16)), full((1, 16)),
                  full((1, 16)), pl.BlockSpec((_NB, _D), lambda i: (i, 0)),
                  full((_D, _D))],
        out_specs=[pl.BlockSpec((_NB, _D), lambda i: (i, 0)), cblk, cblk,
                   cblk],
        out_shape=[jax.ShapeDtypeStruct((_N, _D), jnp.float32)] +
                  [jax.ShapeDtypeStruct((_N, _C), jnp.float32)] * 3,
    )(elements, residues, positions, elem_table, res_table, pos_W, pos_b,
      pe, Wm0)


_T2B = 4096


def _t2_body(cs_ref, cd_ref, w1_ref, b1_ref, w2_ref, b2_ref, cen_ref,
             e0_ref, e1_ref, e2_ref):
    i = pl.program_id(0)
    d = cs_ref[...] - cd_ref[...]
    dist = jnp.sqrt(jnp.sum(d * d, axis=1, keepdims=True) + 1e-12)
    rbf = jnp.exp(-10.0 * (dist - cen_ref[...]) ** 2)
    h1 = jnp.maximum(jnp.dot(rbf, w1_ref[...])
                     + b1_ref[...], 0.0)
    e = jnp.dot(h1, w2_ref[...]) + b2_ref[...]
    rid = i * _T2B + lax.broadcasted_iota(jnp.int32, (_T2B, 1), 0)
    e = jnp.where(rid < _E, e, _NEG)
    e0_ref[...] = e[:, 0:16]
    e1_ref[...] = e[:, 16:32]
    e2_ref[...] = e[:, 32:48]


def _edge_mlp(csrc, cdst, edge_W1, edge_b1, edge_W2, edge_b2, centers):
    full = lambda s: pl.BlockSpec(s, lambda i: tuple(0 for _ in s))
    eb = pl.BlockSpec((_T2B, 16), lambda i: (i, 0))
    return pl.pallas_call(
        _t2_body,
        grid=(_E_PAD // _T2B,),
        in_specs=[eb, eb, full((_EDGE_DIM, _EDGE_HID)), full((1, _EDGE_HID)),
                  full((_EDGE_HID, _D)), full((1, _D)), full((1, _EDGE_DIM))],
        out_specs=[eb, eb, eb],
        out_shape=[jax.ShapeDtypeStruct((_E_PAD, _C), jnp.float32)] * 3,
    )(csrc, cdst, edge_W1, edge_b1, edge_W2, edge_b2, centers)


def _t3_body(last, h_ref, p0_ref, p1_ref, p2_ref, ws_ref, wa_ref, b_ref,
             wn_ref, bo_ref, o_ref, w0_ref, w1_ref, w2_ref):
    agg = jnp.concatenate([p0_ref[0] + p0_ref[1], p1_ref[0] + p1_ref[1],
                           p2_ref[0] + p2_ref[1]], axis=1)
    h = jnp.maximum(jnp.dot(h_ref[...], ws_ref[...])
                    + jnp.dot(agg, wa_ref[...])
                    + b_ref[...], 0.0)
    if last:
        o = jnp.dot(h, wn_ref[...]) + bo_ref[...]
        o_ref[...] = jnp.concatenate(
            [o, jnp.ones((_NB, 1), jnp.float32),
             jnp.zeros((_NB, 13), jnp.float32)], axis=1)
    else:
        o_ref[...] = h
        hw = jnp.dot(h, wn_ref[...])
        w0_ref[...] = hw[:, 0:16]
        w1_ref[...] = hw[:, 16:32]
        w2_ref[...] = hw[:, 32:48]


def _node_update(last, h, parts, Ws_l, Wa_l, bl_l, Wn, bo):
    full = lambda s: pl.BlockSpec(s, lambda i: tuple(0 for _ in s))
    hb = pl.BlockSpec((_NB, _D), lambda i: (i, 0))
    pb = pl.BlockSpec((2, _NB, _C), lambda i: (0, i, 0))
    cblk = pl.BlockSpec((_NB, _C), lambda i: (i, 0))
    if last:
        out_specs = [cblk]
        out_shape = [jax.ShapeDtypeStruct((_N, _C), jnp.float32)]
        wn_spec = full((_D, 2))
        bo_spec = full((1, 2))
    else:
        out_specs = [hb, cblk, cblk, cblk]
        out_shape = ([jax.ShapeDtypeStruct((_N, _D), jnp.float32)] +
                     [jax.ShapeDtypeStruct((_N, _C), jnp.float32)] * 3)
        wn_spec = full((_D, _D))
        bo_spec = full((1, 2))
    body = functools.partial(_t3_body, last)
    if last:
        def body(h_ref, p0, p1, p2, ws, wa, b, wn, bo_, o_ref):
            _t3_body(True, h_ref, p0, p1, p2, ws, wa, b, wn, bo_, o_ref,
                     None, None, None)
    return pl.pallas_call(
        body,
        grid=(_N // _NB,),
        in_specs=[hb, pb, pb, pb, full((_D, _D)), full((_D, _D)),
                  full((1, _D)), wn_spec, bo_spec],
        out_specs=out_specs,
        out_shape=out_shape,
    )(h, parts[0], parts[1], parts[2], Ws_l, Wa_l, bl_l, Wn, bo)


def _t4_body(r_ref, o_ref):
    s = r_ref[0] + r_ref[1]
    o_ref[...] = s[:, 0:2] / jnp.maximum(s[:, 2:3], 1.0)


def _residue_mean(R):
    return pl.pallas_call(
        _t4_body,
        out_shape=jax.ShapeDtypeStruct((_NUM_RES, 2), jnp.float32),
    )(R)


def _k1_body(coords_hbm, src_hbm, dst_hbm, os_hbm, od_hbm,
             idxs, idxd, gs, gd, sem):
    core = lax.axis_index("c")
    sub = lax.axis_index("s")
    tile = core * 16 + sub

    @pl.loop(0, _EBLKS)
    def _blk(b):
        rowbase = tile * _ROWS_PER_TILE + b * 8
        ebase = rowbase * _SUB
        pltpu.sync_copy(src_hbm.at[pl.ds(rowbase, 8)], idxs)
        pltpu.sync_copy(dst_hbm.at[pl.ds(rowbase, 8)], idxd)
        descs = []
        for j in range(8):
            descs.append(pltpu.async_copy(
                coords_hbm.at[idxs.at[j]],
                gs.at[pl.ds(j * _SUB, _SUB)], sem))
        for j in range(8):
            descs.append(pltpu.async_copy(
                coords_hbm.at[idxd.at[j]],
                gd.at[pl.ds(j * _SUB, _SUB)], sem))
        for d in descs:
            d.wait()
        pltpu.sync_copy(gs, os_hbm.at[pl.ds(ebase, _BLK)])
        pltpu.sync_copy(gd, od_hbm.at[pl.ds(ebase, _BLK)])


def _coord_gather(coords_pad, src2d, dst2d):
    f32 = jnp.float32
    return pl.kernel(
        _k1_body,
        out_type=[jax.ShapeDtypeStruct((_E_PAD, 16), f32)] * 2,
        mesh=plsc.VectorSubcoreMesh(**_MESH),
        compiler_params=_SC_PARAMS,
        scratch_types=[
            pltpu.VMEM((8, _SUB), jnp.int32),
            pltpu.VMEM((8, _SUB), jnp.int32),
            pltpu.VMEM((_BLK, 16), f32),
            pltpu.VMEM((_BLK, 16), f32),
            pltpu.SemaphoreType.DMA,
        ],
    )(coords_pad, src2d, dst2d)


_BLK2 = 512
_SB2 = _BLK2 // _SUB
_EBLKS2 = _E_PAD // _NTILES // _BLK2
_ROWS2 = _BLK2 // _SUB * _EBLKS2


def _k2_body(w0, w1, w2, e0, e1, e2, src_hbm, dst_hbm, o0, o1, o2,
             idxs, idxd, g, ebuf, acc, sem):
    core = lax.axis_index("c")
    sub = lax.axis_index("s")
    tile = core * 16 + sub
    zb = sub * 6250

    for w_hbm, e_hbm, o_hbm in ((w0, e0, o0), (w1, e1, o1), (w2, e2, o2)):
        @pl.loop(0, _BLK2)
        def _z(i):
            g[i] = jnp.zeros((16,), jnp.float32)

        for z in range(12):
            pltpu.sync_copy(g, acc.at[pl.ds(zb + z * _BLK2, _BLK2)])
        pltpu.sync_copy(g.at[pl.ds(0, 106)],
                        acc.at[pl.ds(zb + 12 * _BLK2, 106)])
        plsc.subcore_barrier()

        @pl.loop(0, _EBLKS2)
        def _blk(b):
            rowbase = tile * _ROWS2 + b * _SB2
            ebase = rowbase * _SUB
            pltpu.sync_copy(src_hbm.at[pl.ds(rowbase, _SB2)], idxs)
            pltpu.sync_copy(dst_hbm.at[pl.ds(rowbase, _SB2)], idxd)
            descs = []
            for j in range(_SB2):
                descs.append(pltpu.async_copy(
                    w_hbm.at[idxs.at[j]], g.at[pl.ds(j * _SUB, _SUB)], sem))
            pltpu.sync_copy(e_hbm.at[pl.ds(ebase, _BLK2)], ebuf)
            for d in descs:
                d.wait()

            @pl.loop(0, _BLK2, step=8)
            def _row(i):
                for k in range(8):
                    r = i + k
                    g[r] = jnp.maximum(g[r] + ebuf[r], 0.0)

            for j in range(_SB2):
                pltpu.sync_copy(g.at[pl.ds(j * _SUB, _SUB)],
                                acc.at[idxd.at[j]], add=True)

        plsc.subcore_barrier()
        pltpu.sync_copy(acc.at[pl.ds(zb, 6250)],
                        o_hbm.at[core].at[pl.ds(zb, 6250)])
        plsc.subcore_barrier()


def _message_pass(hw_chunks, e_chunks, src2d, dst2d):
    f32 = jnp.float32
    return pl.kernel(
        _k2_body,
        out_type=[jax.ShapeDtypeStruct((2, _N, _C), f32)] * 3,
        mesh=plsc.VectorSubcoreMesh(**_MESH),
        compiler_params=_SC_PARAMS,
        scratch_types=[
            pltpu.VMEM((_SB2, _SUB), jnp.int32),
            pltpu.VMEM((_SB2, _SUB), jnp.int32),
            pltpu.VMEM((_BLK2, 16), f32),
            pltpu.VMEM((_BLK2, 16), f32),
            pltpu.VMEM_SHARED((_N, _C), f32),
            pltpu.SemaphoreType.DMA,
        ],
    )(hw_chunks[0], hw_chunks[1], hw_chunks[2],
      e_chunks[0], e_chunks[1], e_chunks[2], src2d, dst2d)


_K3_ROWS = _RIDROWS // _NTILES


def _k3_body(vals_hbm, rid_hbm, o_hbm, idx, vbuf, zbuf, acc, sem):
    core = lax.axis_index("c")
    sub = lax.axis_index("s")
    tile = core * 16 + sub

    @pl.loop(0, _BLK)
    def _z(i):
        zbuf[i] = jnp.zeros((16,), jnp.float32)

    zb = sub * 313
    nrows = jnp.minimum(jnp.int32(313), jnp.int32(_NUM_RES) - zb)

    @pl.when(sub == 0)
    def _():
        for z in range(4):
            pltpu.sync_copy(zbuf, acc.at[pl.ds(z * _BLK, _BLK)])
        pltpu.sync_copy(zbuf.at[pl.ds(0, _NUM_RES - 4 * _BLK)],
                        acc.at[pl.ds(4 * _BLK, _NUM_RES - 4 * _BLK)])
    del zb, nrows
    plsc.subcore_barrier()

    rowbase = tile * _K3_ROWS
    pltpu.sync_copy(rid_hbm.at[pl.ds(rowbase, _K3_ROWS)], idx)
    pltpu.sync_copy(vals_hbm.at[pl.ds(rowbase * _SUB, _K3_ROWS * _SUB)], vbuf)
    for j in range(_K3_ROWS):
        pltpu.sync_copy(vbuf.at[pl.ds(j * _SUB, _SUB)],
                        acc.at[idx.at[j]], add=True)
    plsc.subcore_barrier()

    @pl.when(sub == 0)
    def _():
        pltpu.sync_copy(acc, o_hbm.at[core])


def _residue_scatter(vals_pad, rid2d):
    f32 = jnp.float32
    return pl.kernel(
        _k3_body,
        out_type=jax.ShapeDtypeStruct((2, _NUM_RES, _C), f32),
        mesh=plsc.VectorSubcoreMesh(**_MESH),
        compiler_params=_SC_PARAMS,
        scratch_types=[
            pltpu.VMEM((_K3_ROWS, _SUB), jnp.int32),
            pltpu.VMEM((_K3_ROWS * _SUB, 16), f32),
            pltpu.VMEM((_BLK, 16), f32),
            pltpu.VMEM_SHARED((_NUM_RES, _C), f32),
            pltpu.SemaphoreType.DMA,
        ],
    )(vals_pad, rid2d)


def kernel(coordinates, positions, elem_table, res_table, pos_W, pos_b,
           edge_W1, edge_b1, edge_W2, edge_b2, Wm, Ws, Wa, bl, Wo, bo,
           elements, residues, edge_index, residue_ids):
    i32 = jnp.int32
    f32 = jnp.float32
    src = edge_index[0].astype(i32)
    dst = edge_index[1].astype(i32)
    src2d = jnp.pad(src, (0, _E_PAD - _E)).reshape(_IDXROWS, _SUB)
    dst2d = jnp.pad(dst, (0, _E_PAD - _E)).reshape(_IDXROWS, _SUB)
    coords_pad = jnp.pad(coordinates.astype(f32), ((0, 0), (0, 13)))
    pe = _sinusoidal(_N, _D)
    centers = jnp.linspace(0.0, 10.0, _EDGE_DIM).reshape(1, _EDGE_DIM)

    csrc, cdst = _coord_gather(coords_pad, src2d, dst2d)

    h, w0, w1, w2 = _node_features(
        elements.astype(i32).reshape(_N, 1), residues.astype(i32).reshape(_N, 1),
        positions.astype(f32).reshape(_N, 1), elem_table, res_table,
        pos_W, pos_b.reshape(1, _HIDDEN), pe, Wm[0])

    e_chunks = _edge_mlp(csrc, cdst, edge_W1, edge_b1.reshape(1, _EDGE_HID),
                         edge_W2, edge_b2.reshape(1, _D), centers)

    parts = _message_pass((w0, w1, w2), e_chunks, src2d, dst2d)
    h, w0, w1, w2 = _node_update(False, h, parts, Ws[0], Wa[0],
                                 bl[0].reshape(1, _D), Wm[1],
                                 bo.reshape(1, 2))

    parts = _message_pass((w0, w1, w2), e_chunks, src2d, dst2d)
    (outp,) = (_node_update(True, h, parts, Ws[1], Wa[1],
                            bl[1].reshape(1, _D), Wo, bo.reshape(1, 2)),)
    outp = outp[0] if isinstance(outp, (list, tuple)) else outp

    vals_pad = jnp.pad(outp, ((0, _N_PAD - _N), (0, 0)))
    rid2d = jnp.pad(residue_ids.astype(i32),
                    (0, _N_PAD - _N)).reshape(_RIDROWS, _SUB)
    R = _residue_scatter(vals_pad, rid2d)

    return _residue_mean(R)


# speedup vs baseline: 5.3383x; 1.6418x over previous
import functools

import jax
import jax.numpy as jnp
from jax import lax
from jax.experimental import pallas as pl
from jax.experimental.pallas import tpu as pltpu
from jax.experimental.pallas import tpu_sc as plsc

_N = 100000
_E = 1600000
_HIDDEN = 16
_D = 48
_EDGE_DIM = 16
_EDGE_HID = 32
_NUM_RES = 5000

_C = 16
_NCHUNK = 3
_NTILES = 32
_BLK = 1024
_SUB = 128
_EBLKS = 49
_E_PAD = _NTILES * _BLK * _EBLKS
_IDXROWS = _E_PAD // _SUB
_ROWS_PER_TILE = _IDXROWS // _NTILES
_NB = 800
_N_PAD = 102400
_RIDROWS = _N_PAD // _SUB

_MESH = dict(core_axis_name="c", subcore_axis_name="s", num_cores=2,
             num_subcores=16)
_SC_PARAMS = pltpu.CompilerParams(use_tc_tiling_on_sc=False)
_NEG = -1e30


def _sinusoidal(n, dim):
    pos = jnp.arange(n, dtype=jnp.float32)[:, None]
    i = jnp.arange(dim // 2, dtype=jnp.float32)[None, :]
    div = jnp.exp(-jnp.log(10000.0) * (2.0 * i) / dim)
    ang = pos * div
    return jnp.stack([jnp.sin(ang), jnp.cos(ang)], axis=-1).reshape(n, dim)


def _t1_body(el_ref, rs_ref, ps_ref, et_ref, rt_ref, pw_ref, pb_ref, pe_ref,
             wm_ref, h_ref, w0_ref, w1_ref, w2_ref):
    el = el_ref[...]
    rs = rs_ref[...]
    e_emb = jnp.zeros((_NB, _HIDDEN), jnp.float32)
    for k in range(5):
        e_emb = e_emb + jnp.where(el == k, 1.0, 0.0) * et_ref[k:k + 1, :]
    r_emb = jnp.zeros((_NB, _HIDDEN), jnp.float32)
    for k in range(4):
        r_emb = r_emb + jnp.where(rs == k, 1.0, 0.0) * rt_ref[k:k + 1, :]
    p_emb = ps_ref[...] * pw_ref[...] + pb_ref[...]
    h = jnp.concatenate([e_emb, r_emb, p_emb], axis=1) + pe_ref[...]
    h_ref[...] = h
    hw = jnp.dot(h, wm_ref[...])
    w0_ref[...] = hw[:, 0:16]
    w1_ref[...] = hw[:, 16:32]
    w2_ref[...] = hw[:, 32:48]


def _node_features(elements, residues, positions, elem_table, res_table,
                   pos_W, pos_b, pe, Wm0):
    nb = pl.BlockSpec((_NB, 1), lambda i: (i, 0))
    full = lambda s: pl.BlockSpec(s, lambda i: tuple(0 for _ in s))
    cblk = pl.BlockSpec((_NB, _C), lambda i: (i, 0))
    return pl.pallas_call(
        _t1_body,
        grid=(_N // _NB,),
        in_specs=[nb, nb, nb, full((5, 16)), full((4, 16)), full((1, 16)),
                  full((1, 16)), pl.BlockSpec((_NB, _D), lambda i: (i, 0)),
                  full((_D, _D))],
        out_specs=[pl.BlockSpec((_NB, _D), lambda i: (i, 0)), cblk, cblk,
                   cblk],
        out_shape=[jax.ShapeDtypeStruct((_N, _D), jnp.float32)] +
                  [jax.ShapeDtypeStruct((_N, _C), jnp.float32)] * 3,
    )(elements, residues, positions, elem_table, res_table, pos_W, pos_b,
      pe, Wm0)


_T2B = 4096
_T2R = _T2B // 8
_EROWS = _E_PAD // 8


def _tile8(w):
    r = jnp.concatenate([w] * 8, axis=0)
    return jnp.concatenate([r] * 8, axis=1)


def _bdiag(w):
    a, b = w.shape
    t = _tile8(w)
    ri = lax.broadcasted_iota(jnp.int32, (8 * a, 8 * b), 0) // a
    ci = lax.broadcasted_iota(jnp.int32, (8 * a, 8 * b), 1) // b
    return jnp.where(ri == ci, t, 0.0)


def _t2_body(cs_ref, cd_ref, w1_ref, b1_ref, w2_ref, b2_ref, cen_ref,
             e0_ref, e1_ref, e2_ref):
    i = pl.program_id(0)
    d = cs_ref[...] - cd_ref[...]
    d2 = d * d
    bi = lax.broadcasted_iota(jnp.int32, (128, 128), 0) // 16
    bj = lax.broadcasted_iota(jnp.int32, (128, 128), 1) // 16
    gsum = jnp.where(bi == bj, 1.0, 0.0)
    dist2 = jnp.dot(d2, gsum, precision=jax.lax.Precision.HIGHEST)
    dist = jnp.sqrt(dist2 + 1e-12)
    cen = jnp.concatenate([cen_ref[...]] * 8, axis=1)
    rbf = jnp.exp(-10.0 * (dist - cen) ** 2)
    w1bd = _bdiag(w1_ref[...])
    w2bd = _bdiag(w2_ref[...])
    b1t = jnp.concatenate([b1_ref[...]] * 8, axis=1)
    b2t = jnp.concatenate([b2_ref[...]] * 8, axis=1)
    h1 = jnp.maximum(jnp.dot(rbf, w1bd) + b1t, 0.0)
    e = jnp.dot(h1, w2bd) + b2t
    rr = lax.broadcasted_iota(jnp.int32, (_T2R, 384), 0)
    cc = lax.broadcasted_iota(jnp.int32, (_T2R, 384), 1)
    eid = i * _T2B + rr * 8 + cc // _D
    e = jnp.where(eid < _E, e, _NEG)
    e0_ref[...] = jnp.concatenate(
        [e[:, 48 * k:48 * k + 16] for k in range(8)], axis=1)
    e1_ref[...] = jnp.concatenate(
        [e[:, 48 * k + 16:48 * k + 32] for k in range(8)], axis=1)
    e2_ref[...] = jnp.concatenate(
        [e[:, 48 * k + 32:48 * k + 48] for k in range(8)], axis=1)


def _edge_mlp(csrc_p, cdst_p, edge_W1, edge_b1, edge_W2, edge_b2, centers):
    full = lambda s: pl.BlockSpec(s, lambda i: tuple(0 for _ in s))
    eb = pl.BlockSpec((_T2R, 128), lambda i: (i, 0))
    return pl.pallas_call(
        _t2_body,
        grid=(_EROWS // _T2R,),
        in_specs=[eb, eb, full((_EDGE_DIM, _EDGE_HID)), full((1, _EDGE_HID)),
                  full((_EDGE_HID, _D)), full((1, _D)), full((1, _EDGE_DIM))],
        out_specs=[eb, eb, eb],
        out_shape=[jax.ShapeDtypeStruct((_EROWS, 128), jnp.float32)] * 3,
    )(csrc_p, cdst_p, edge_W1, edge_b1, edge_W2, edge_b2, centers)


def _t3_body(last, h_ref, p0_ref, p1_ref, p2_ref, ws_ref, wa_ref, b_ref,
             wn_ref, bo_ref, o_ref, w0_ref, w1_ref, w2_ref):
    agg = jnp.concatenate([p0_ref[0] + p0_ref[1], p1_ref[0] + p1_ref[1],
                           p2_ref[0] + p2_ref[1]], axis=1)
    h = jnp.maximum(jnp.dot(h_ref[...], ws_ref[...])
                    + jnp.dot(agg, wa_ref[...])
                    + b_ref[...], 0.0)
    if last:
        o = jnp.dot(h, wn_ref[...]) + bo_ref[...]
        o_ref[...] = jnp.concatenate(
            [o, jnp.ones((_NB, 1), jnp.float32),
             jnp.zeros((_NB, 13), jnp.float32)], axis=1)
    else:
        o_ref[...] = h
        hw = jnp.dot(h, wn_ref[...])
        w0_ref[...] = hw[:, 0:16]
        w1_ref[...] = hw[:, 16:32]
        w2_ref[...] = hw[:, 32:48]


def _node_update(last, h, parts, Ws_l, Wa_l, bl_l, Wn, bo):
    full = lambda s: pl.BlockSpec(s, lambda i: tuple(0 for _ in s))
    hb = pl.BlockSpec((_NB, _D), lambda i: (i, 0))
    pb = pl.BlockSpec((2, _NB, _C), lambda i: (0, i, 0))
    cblk = pl.BlockSpec((_NB, _C), lambda i: (i, 0))
    if last:
        out_specs = [cblk]
        out_shape = [jax.ShapeDtypeStruct((_N, _C), jnp.float32)]
        wn_spec = full((_D, 2))
        bo_spec = full((1, 2))
    else:
        out_specs = [hb, cblk, cblk, cblk]
        out_shape = ([jax.ShapeDtypeStruct((_N, _D), jnp.float32)] +
                     [jax.ShapeDtypeStruct((_N, _C), jnp.float32)] * 3)
        wn_spec = full((_D, _D))
        bo_spec = full((1, 2))
    body = functools.partial(_t3_body, last)
    if last:
        def body(h_ref, p0, p1, p2, ws, wa, b, wn, bo_, o_ref):
            _t3_body(True, h_ref, p0, p1, p2, ws, wa, b, wn, bo_, o_ref,
                     None, None, None)
    return pl.pallas_call(
        body,
        grid=(_N // _NB,),
        in_specs=[hb, pb, pb, pb, full((_D, _D)), full((_D, _D)),
                  full((1, _D)), wn_spec, bo_spec],
        out_specs=out_specs,
        out_shape=out_shape,
    )(h, parts[0], parts[1], parts[2], Ws_l, Wa_l, bl_l, Wn, bo)


def _t4_body(r_ref, o_ref):
    s = r_ref[0] + r_ref[1]
    o_ref[...] = s[:, 0:2] / jnp.maximum(s[:, 2:3], 1.0)


def _residue_mean(R):
    return pl.pallas_call(
        _t4_body,
        out_shape=jax.ShapeDtypeStruct((_NUM_RES, 2), jnp.float32),
    )(R)


def _k1_body(coords_hbm, src_hbm, dst_hbm, os_hbm, od_hbm,
             idxs, idxd, gs, gd, sem):
    core = lax.axis_index("c")
    sub = lax.axis_index("s")
    tile = core * 16 + sub

    @pl.loop(0, _EBLKS)
    def _blk(b):
        rowbase = tile * _ROWS_PER_TILE + b * 8
        ebase = rowbase * _SUB
        pltpu.sync_copy(src_hbm.at[pl.ds(rowbase, 8)], idxs)
        pltpu.sync_copy(dst_hbm.at[pl.ds(rowbase, 8)], idxd)
        descs = []
        for j in range(8):
            descs.append(pltpu.async_copy(
                coords_hbm.at[idxs.at[j]],
                gs.at[pl.ds(j * _SUB, _SUB)], sem))
        for j in range(8):
            descs.append(pltpu.async_copy(
                coords_hbm.at[idxd.at[j]],
                gd.at[pl.ds(j * _SUB, _SUB)], sem))
        for d in descs:
            d.wait()
        pltpu.sync_copy(gs, os_hbm.at[pl.ds(ebase, _BLK)])
        pltpu.sync_copy(gd, od_hbm.at[pl.ds(ebase, _BLK)])


def _coord_gather(coords_pad, src2d, dst2d):
    f32 = jnp.float32
    return pl.kernel(
        _k1_body,
        out_type=[jax.ShapeDtypeStruct((_E_PAD, 16), f32)] * 2,
        mesh=plsc.VectorSubcoreMesh(**_MESH),
        compiler_params=_SC_PARAMS,
        scratch_types=[
            pltpu.VMEM((8, _SUB), jnp.int32),
            pltpu.VMEM((8, _SUB), jnp.int32),
            pltpu.VMEM((_BLK, 16), f32),
            pltpu.VMEM((_BLK, 16), f32),
            pltpu.SemaphoreType.DMA,
        ],
    )(coords_pad, src2d, dst2d)


_BLK2 = 512
_SB2 = _BLK2 // _SUB
_EBLKS2 = _E_PAD // _NTILES // _BLK2
_ROWS2 = _BLK2 // _SUB * _EBLKS2


def _k2_body(w0, w1, w2, e0, e1, e2, src_hbm, dst_hbm, o0, o1, o2,
             idxs, idxd, g, ebuf, acc, sem):
    core = lax.axis_index("c")
    sub = lax.axis_index("s")
    tile = core * 16 + sub
    zb = sub * 6250

    for w_hbm, e_hbm, o_hbm in ((w0, e0, o0), (w1, e1, o1), (w2, e2, o2)):
        @pl.loop(0, _BLK2)
        def _z(i):
            g[i] = jnp.zeros((16,), jnp.float32)

        for z in range(12):
            pltpu.sync_copy(g, acc.at[pl.ds(zb + z * _BLK2, _BLK2)])
        pltpu.sync_copy(g.at[pl.ds(0, 106)],
                        acc.at[pl.ds(zb + 12 * _BLK2, 106)])
        plsc.subcore_barrier()

        @pl.loop(0, _EBLKS2)
        def _blk(b):
            rowbase = tile * _ROWS2 + b * _SB2
            ebase = rowbase * _SUB
            pltpu.sync_copy(src_hbm.at[pl.ds(rowbase, _SB2)], idxs)
            pltpu.sync_copy(dst_hbm.at[pl.ds(rowbase, _SB2)], idxd)
            descs = []
            for j in range(_SB2):
                descs.append(pltpu.async_copy(
                    w_hbm.at[idxs.at[j]], g.at[pl.ds(j * _SUB, _SUB)], sem))
            pltpu.sync_copy(e_hbm.at[pl.ds(ebase, _BLK2)], ebuf)
            for d in descs:
                d.wait()

            @pl.loop(0, _BLK2, step=8)
            def _row(i):
                for k in range(8):
                    r = i + k
                    g[r] = jnp.maximum(g[r] + ebuf[r], 0.0)

            for j in range(_SB2):
                pltpu.sync_copy(g.at[pl.ds(j * _SUB, _SUB)],
                                acc.at[idxd.at[j]], add=True)

        plsc.subcore_barrier()
        pltpu.sync_copy(acc.at[pl.ds(zb, 6250)],
                        o_hbm.at[core].at[pl.ds(zb, 6250)])
        plsc.subcore_barrier()


def _message_pass(hw_chunks, e_chunks, src2d, dst2d):
    f32 = jnp.float32
    return pl.kernel(
        _k2_body,
        out_type=[jax.ShapeDtypeStruct((2, _N, _C), f32)] * 3,
        mesh=plsc.VectorSubcoreMesh(**_MESH),
        compiler_params=_SC_PARAMS,
        scratch_types=[
            pltpu.VMEM((_SB2, _SUB), jnp.int32),
            pltpu.VMEM((_SB2, _SUB), jnp.int32),
            pltpu.VMEM((_BLK2, 16), f32),
            pltpu.VMEM((_BLK2, 16), f32),
            pltpu.VMEM_SHARED((_N, _C), f32),
            pltpu.SemaphoreType.DMA,
        ],
    )(hw_chunks[0], hw_chunks[1], hw_chunks[2],
      e_chunks[0], e_chunks[1], e_chunks[2], src2d, dst2d)


_K3_ROWS = _RIDROWS // _NTILES


def _k3_body(vals_hbm, rid_hbm, o_hbm, idx, vbuf, zbuf, acc, sem):
    core = lax.axis_index("c")
    sub = lax.axis_index("s")
    tile = core * 16 + sub

    @pl.loop(0, _BLK)
    def _z(i):
        zbuf[i] = jnp.zeros((16,), jnp.float32)

    zb = sub * 313
    nrows = jnp.minimum(jnp.int32(313), jnp.int32(_NUM_RES) - zb)

    @pl.when(sub == 0)
    def _():
        for z in range(4):
            pltpu.sync_copy(zbuf, acc.at[pl.ds(z * _BLK, _BLK)])
        pltpu.sync_copy(zbuf.at[pl.ds(0, _NUM_RES - 4 * _BLK)],
                        acc.at[pl.ds(4 * _BLK, _NUM_RES - 4 * _BLK)])
    del zb, nrows
    plsc.subcore_barrier()

    rowbase = tile * _K3_ROWS
    pltpu.sync_copy(rid_hbm.at[pl.ds(rowbase, _K3_ROWS)], idx)
    pltpu.sync_copy(vals_hbm.at[pl.ds(rowbase * _SUB, _K3_ROWS * _SUB)], vbuf)
    for j in range(_K3_ROWS):
        pltpu.sync_copy(vbuf.at[pl.ds(j * _SUB, _SUB)],
                        acc.at[idx.at[j]], add=True)
    plsc.subcore_barrier()

    @pl.when(sub == 0)
    def _():
        pltpu.sync_copy(acc, o_hbm.at[core])


def _residue_scatter(vals_pad, rid2d):
    f32 = jnp.float32
    return pl.kernel(
        _k3_body,
        out_type=jax.ShapeDtypeStruct((2, _NUM_RES, _C), f32),
        mesh=plsc.VectorSubcoreMesh(**_MESH),
        compiler_params=_SC_PARAMS,
        scratch_types=[
            pltpu.VMEM((_K3_ROWS, _SUB), jnp.int32),
            pltpu.VMEM((_K3_ROWS * _SUB, 16), f32),
            pltpu.VMEM((_BLK, 16), f32),
            pltpu.VMEM_SHARED((_NUM_RES, _C), f32),
            pltpu.SemaphoreType.DMA,
        ],
    )(vals_pad, rid2d)


def kernel(coordinates, positions, elem_table, res_table, pos_W, pos_b,
           edge_W1, edge_b1, edge_W2, edge_b2, Wm, Ws, Wa, bl, Wo, bo,
           elements, residues, edge_index, residue_ids):
    i32 = jnp.int32
    f32 = jnp.float32
    src = edge_index[0].astype(i32)
    dst = edge_index[1].astype(i32)
    src2d = jnp.pad(src, (0, _E_PAD - _E)).reshape(_IDXROWS, _SUB)
    dst2d = jnp.pad(dst, (0, _E_PAD - _E)).reshape(_IDXROWS, _SUB)
    coords_pad = jnp.pad(coordinates.astype(f32), ((0, 0), (0, 13)))
    pe = _sinusoidal(_N, _D)
    centers = jnp.linspace(0.0, 10.0, _EDGE_DIM).reshape(1, _EDGE_DIM)

    csrc, cdst = _coord_gather(coords_pad, src2d, dst2d)
    csrc_p = csrc.reshape(_EROWS, 128)
    cdst_p = cdst.reshape(_EROWS, 128)

    h, w0, w1, w2 = _node_features(
        elements.astype(i32).reshape(_N, 1), residues.astype(i32).reshape(_N, 1),
        positions.astype(f32).reshape(_N, 1), elem_table, res_table,
        pos_W, pos_b.reshape(1, _HIDDEN), pe, Wm[0])

    e_packed = _edge_mlp(csrc_p, cdst_p, edge_W1,
                         edge_b1.reshape(1, _EDGE_HID), edge_W2,
                         edge_b2.reshape(1, _D), centers)
    e_chunks = [e.reshape(_E_PAD, _C) for e in e_packed]

    parts = _message_pass((w0, w1, w2), e_chunks, src2d, dst2d)
    h, w0, w1, w2 = _node_update(False, h, parts, Ws[0], Wa[0],
                                 bl[0].reshape(1, _D), Wm[1],
                                 bo.reshape(1, 2))

    parts = _message_pass((w0, w1, w2), e_chunks, src2d, dst2d)
    (outp,) = (_node_update(True, h, parts, Ws[1], Wa[1],
                            bl[1].reshape(1, _D), Wo, bo.reshape(1, 2)),)
    outp = outp[0] if isinstance(outp, (list, tuple)) else outp

    vals_pad = jnp.pad(outp, ((0, _N_PAD - _N), (0, 0)))
    rid2d = jnp.pad(residue_ids.astype(i32),
                    (0, _N_PAD - _N)).reshape(_RIDROWS, _SUB)
    R = _residue_scatter(vals_pad, rid2d)

    return _residue_mean(R)
